# Initial kernel scaffold; baseline (speedup 1.0000x reference)
import jax
import jax.numpy as jnp
from jax.experimental import pallas as pl


def kernel(loc_data, conf_data, landm_data, priors, targets):
    raise NotImplementedError("write your pallas kernel here")


# speedup vs baseline: 24.0949x; 24.0949x over previous
import functools

import jax
import jax.numpy as jnp
from jax.experimental import pallas as pl

_NUM_CLASSES = 3
_THRESHOLD = 0.35
_NEG_POS_RATIO = 7.0
_V0 = 0.1
_V1 = 0.2


def _body(P, loc_ref, conf_ref, landm_ref, priors_ref, tgt_ref, out_ref):
    i = pl.program_id(0)
    PP = priors_ref.shape[1]
    f32 = jnp.float32

    t = tgt_ref[...]
    tx1 = t[:, 0:1]
    ty1 = t[:, 1:2]
    tx2 = t[:, 2:3]
    ty2 = t[:, 3:4]
    label = t[:, 14:15]

    pcx = priors_ref[0:1, :]
    pcy = priors_ref[1:2, :]
    pw = priors_ref[2:3, :]
    ph = priors_ref[3:4, :]
    px1 = pcx - pw * 0.5
    py1 = pcy - ph * 0.5
    px2 = pcx + pw * 0.5
    py2 = pcy + ph * 0.5

    iw = jnp.maximum(jnp.minimum(tx2, px2) - jnp.maximum(tx1, px1), 0.0)
    ih = jnp.maximum(jnp.minimum(ty2, py2) - jnp.maximum(ty1, py1), 0.0)
    inter = iw * ih
    area_a = (tx2 - tx1) * (ty2 - ty1)
    area_b = pw * ph
    lane8 = jax.lax.broadcasted_iota(jnp.int32, (8, PP), 1)
    ov = inter / (area_a + area_b - inter)
    ov = jnp.where(lane8 < P, ov, 0.0)

    bpo = jnp.max(ov, axis=1, keepdims=True)
    bpi = jnp.min(jnp.where(ov == bpo, lane8, PP), axis=1, keepdims=True)
    valid = (bpo >= 0.2).astype(f32)

    sub8 = jax.lax.broadcasted_iota(jnp.int32, (8, PP), 0)
    bto = jnp.max(ov, axis=0, keepdims=True)
    bti = jnp.min(jnp.where(ov == bto, sub8, 8), axis=0, keepdims=True)

    lane1 = lane8[0:1, :]
    bto_orig = bto
    for j in range(8):
        bpi_j = bpi[j : j + 1, 0:1]
        mask_j = lane1 == bpi_j
        g_j = jnp.sum(jnp.where(mask_j, bto_orig, 0.0))
        fill_j = jnp.where(valid[j : j + 1, 0:1] > 0.0, 2.0, g_j)
        bto = jnp.where(mask_j, fill_j, bto)
        bti = jnp.where(mask_j, j, bti)

    ohf = (bti == sub8).astype(f32)

    def gat(vals):
        return jnp.sum(ohf * vals, axis=0, keepdims=True)

    any_valid = jnp.max(valid) > 0.0

    conf = gat(label)
    conf = jnp.where(bto < _THRESHOLD, 0.0, conf)
    conf = jnp.where(any_valid, conf, 0.0)

    m_x1 = gat(tx1)
    m_y1 = gat(ty1)
    m_x2 = gat(tx2)
    m_y2 = gat(ty2)
    lt = [
        ((m_x1 + m_x2) * 0.5 - pcx) / (_V0 * pw),
        ((m_y1 + m_y2) * 0.5 - pcy) / (_V0 * ph),
        jnp.log((m_x2 - m_x1) / pw) / _V1,
        jnp.log((m_y2 - m_y1) / ph) / _V1,
    ]
    lmt = []
    for kp in range(5):
        lmx = gat(t[:, 4 + 2 * kp : 5 + 2 * kp])
        lmy = gat(t[:, 5 + 2 * kp : 6 + 2 * kp])
        lmt.append((lmx - pcx) / (_V0 * pw))
        lmt.append((lmy - pcy) / (_V0 * ph))

    posf = (conf > 0.0).astype(f32)
    facef = (conf == 1.0).astype(f32)
    npos = jnp.sum(posf)

    def sl1(a, b):
        d = jnp.abs(a - b)
        return jnp.where(d < 1.0, 0.5 * d * d, d - 0.5)

    zero = jnp.float32(0.0)
    ll = zero
    for c in range(4):
        v = jnp.where(any_valid, lt[c], 0.0)
        ll = ll + jnp.sum(posf * sl1(loc_ref[c : c + 1, :], v))
    llm = zero
    for c in range(10):
        m = posf if c < 4 else facef
        v = jnp.where(any_valid, lmt[c], 0.0)
        llm = llm + jnp.sum(m * sl1(landm_ref[c : c + 1, :], v))

    c0 = conf_ref[0:1, :]
    c1 = conf_ref[1:2, :]
    c2 = conf_ref[2:3, :]
    mx = jnp.maximum(jnp.maximum(c0, c1), c2)
    lse = mx + jnp.log(jnp.exp(c0 - mx) + jnp.exp(c1 - mx) + jnp.exp(c2 - mx))
    gathered = jnp.where(conf == 0.0, c0, jnp.where(conf == 1.0, c1, c2))
    ce = lse - gathered
    sum_pos_ce = jnp.sum(posf * ce)

    x = jnp.where((posf > 0.0) | (lane1 >= P), 0.0, ce)
    xb = jax.lax.bitcast_convert_type(x, jnp.int32)

    k = jnp.minimum(_NEG_POS_RATIO * npos, f32(P - 1))
    k_eff = jnp.maximum(k, 1.0)

    def bis(_, carry):
        lo, hi = carry
        mid = lo + (hi - lo) // 2
        cnt = jnp.sum((xb >= mid).astype(f32))
        ge = cnt >= k_eff
        return (jnp.where(ge, mid, lo), jnp.where(ge, hi, mid))

    lo, hi = jax.lax.fori_loop(
        0, 32, bis, (jnp.int32(0), jnp.int32(0x7F800000))
    )
    gt = xb > lo
    cnt_gt = jnp.sum(gt.astype(f32))
    sum_gt = jnp.sum(jnp.where(gt, x, 0.0))
    t_val = jnp.max(jnp.where(xb == lo, x, 0.0))
    lc = sum_pos_ce + sum_gt + (k - cnt_gt) * t_val

    sub128 = jax.lax.broadcasted_iota(jnp.int32, (8, 128), 0)
    buf = jnp.where(
        sub128 == 0,
        ll,
        jnp.where(
            sub128 == 1,
            lc,
            jnp.where(sub128 == 2, llm, jnp.where(sub128 == 3, npos, 0.0)),
        ),
    )

    @pl.when(i == 0)
    def _():
        out_ref[...] = jnp.zeros((8, 128), f32)

    out_ref[...] += buf


def kernel(loc_data, conf_data, landm_data, priors, targets):
    num, P, _ = loc_data.shape
    PP = ((P + 127) // 128) * 128
    pad = PP - P
    f32 = jnp.float32

    locT = jnp.pad(loc_data.transpose(0, 2, 1), ((0, 0), (0, 0), (0, pad)))
    confT = jnp.pad(conf_data.transpose(0, 2, 1), ((0, 0), (0, 0), (0, pad)))
    landmT = jnp.pad(landm_data.transpose(0, 2, 1), ((0, 0), (0, 0), (0, pad)))
    pad_cols = jnp.concatenate(
        [jnp.zeros((2, pad), f32), jnp.ones((2, pad), f32)], axis=0
    )
    priorsT = jnp.concatenate([priors.T, pad_cols], axis=1)
    tgt = jnp.pad(targets, ((0, 0), (0, 0), (0, 1)))

    out = pl.pallas_call(
        functools.partial(_body, P),
        grid=(num,),
        in_specs=[
            pl.BlockSpec((None, 4, PP), lambda i: (i, 0, 0)),
            pl.BlockSpec((None, _NUM_CLASSES, PP), lambda i: (i, 0, 0)),
            pl.BlockSpec((None, 10, PP), lambda i: (i, 0, 0)),
            pl.BlockSpec((4, PP), lambda i: (0, 0)),
            pl.BlockSpec((None, 8, 16), lambda i: (i, 0, 0)),
        ],
        out_specs=pl.BlockSpec((8, 128), lambda i: (0, 0)),
        out_shape=jax.ShapeDtypeStruct((8, 128), f32),
    )(locT, confT, landmT, priorsT, tgt)

    ll = out[0, 0]
    lc = out[1, 0]
    llm = out[2, 0]
    npos = out[3, 0]
    n = jnp.maximum(npos, 1.0)
    return ll / n, lc / n, llm / n


# speedup vs baseline: 97.0020x; 4.0258x over previous
import functools

import jax
import jax.numpy as jnp
from jax.experimental import pallas as pl

_NUM_CLASSES = 3
_THRESHOLD = 0.35
_NEG_POS_RATIO = 7.0
_V0 = 0.1
_V1 = 0.2


def _body(P, loc_ref, conf_ref, landm_ref, priors_ref, tgt_ref, out_ref):
    i = pl.program_id(0)
    PP = priors_ref.shape[1]
    f32 = jnp.float32

    tgt = tgt_ref[...]

    pcx = priors_ref[0:1, :]
    pcy = priors_ref[1:2, :]
    pw = priors_ref[2:3, :]
    ph = priors_ref[3:4, :]
    px1 = pcx - pw * 0.5
    py1 = pcy - ph * 0.5
    px2 = pcx + pw * 0.5
    py2 = pcy + ph * 0.5

    lane = jax.lax.broadcasted_iota(jnp.int32, (8, PP), 1)
    in_range = lane < P

    def col(j, c):
        return tgt[:, j, c : c + 1]

    bto = None
    bti = None
    bpo = []
    bpi = []
    for j in range(8):
        tx1 = col(j, 0)
        ty1 = col(j, 1)
        tx2 = col(j, 2)
        ty2 = col(j, 3)
        iw = jnp.maximum(jnp.minimum(tx2, px2) - jnp.maximum(tx1, px1), 0.0)
        ih = jnp.maximum(jnp.minimum(ty2, py2) - jnp.maximum(ty1, py1), 0.0)
        inter = iw * ih
        area_a = (tx2 - tx1) * (ty2 - ty1)
        ov = inter / (area_a + pw * ph - inter)
        ov = jnp.where(in_range, ov, 0.0)
        mx = jnp.max(ov, axis=1, keepdims=True)
        bpo.append(mx)
        bpi.append(
            jnp.min(jnp.where(ov == mx, lane, PP), axis=1, keepdims=True)
        )
        if j == 0:
            bto = ov
            bti = jnp.zeros((8, PP), jnp.int32)
        else:
            upd = ov > bto
            bto = jnp.where(upd, ov, bto)
            bti = jnp.where(upd, j, bti)

    bto_orig = bto
    any_v = jnp.zeros((8, 1), f32)
    for j in range(8):
        valid_j = (bpo[j] >= 0.2).astype(f32)
        any_v = jnp.maximum(any_v, valid_j)
        mask_j = lane == bpi[j]
        g_j = jnp.sum(jnp.where(mask_j, bto_orig, 0.0), axis=1, keepdims=True)
        fill_j = jnp.where(valid_j > 0.0, 2.0, g_j)
        bto = jnp.where(mask_j, fill_j, bto)
        bti = jnp.where(mask_j, j, bti)

    cols = [0, 1, 2, 3, 14] + list(range(4, 14))
    nco = len(cols)
    acc = [None] * nco
    for j in range(8):
        if j == 0:
            for a in range(nco):
                acc[a] = jnp.broadcast_to(col(0, cols[a]), (8, PP))
        else:
            mask_j = bti == j
            for a in range(nco):
                acc[a] = jnp.where(mask_j, col(j, cols[a]), acc[a])
    m_x1, m_y1, m_x2, m_y2, label = acc[:5]
    lmc = acc[5:]

    conf = jnp.where(bto < _THRESHOLD, 0.0, label)
    conf = jnp.where(any_v > 0.0, conf, 0.0)

    lt = [
        ((m_x1 + m_x2) * 0.5 - pcx) / (_V0 * pw),
        ((m_y1 + m_y2) * 0.5 - pcy) / (_V0 * ph),
        jnp.log((m_x2 - m_x1) / pw) / _V1,
        jnp.log((m_y2 - m_y1) / ph) / _V1,
    ]
    lmt = []
    for kp in range(5):
        lmt.append((lmc[2 * kp] - pcx) / (_V0 * pw))
        lmt.append((lmc[2 * kp + 1] - pcy) / (_V0 * ph))

    posf = (conf > 0.0).astype(f32)
    facef = (conf == 1.0).astype(f32)
    npos_row = jnp.sum(posf, axis=1, keepdims=True)

    def sl1(a, b):
        d = jnp.abs(a - b)
        return jnp.where(d < 1.0, 0.5 * d * d, d - 0.5)

    ll = jnp.float32(0.0)
    for c in range(4):
        ll = ll + jnp.sum(posf * sl1(loc_ref[c], lt[c]))
    llm = jnp.float32(0.0)
    for c in range(10):
        m = posf if c < 4 else facef
        llm = llm + jnp.sum(m * sl1(landm_ref[c], lmt[c]))

    c0 = conf_ref[0]
    c1 = conf_ref[1]
    c2 = conf_ref[2]
    mx3 = jnp.maximum(jnp.maximum(c0, c1), c2)
    lse = mx3 + jnp.log(
        jnp.exp(c0 - mx3) + jnp.exp(c1 - mx3) + jnp.exp(c2 - mx3)
    )
    gathered = jnp.where(conf == 0.0, c0, jnp.where(conf == 1.0, c1, c2))
    ce = lse - gathered
    sum_pos_ce = jnp.sum(posf * ce, axis=1, keepdims=True)

    x = jnp.where((posf > 0.0) | jnp.logical_not(in_range), 0.0, ce)
    xb = jax.lax.bitcast_convert_type(x, jnp.int32)

    k = jnp.minimum(_NEG_POS_RATIO * npos_row, f32(P - 1))
    k_eff = jnp.maximum(k, 1.0)

    def bis(_, carry):
        lo, hi = carry
        mid = lo + (hi - lo) // 2
        cnt = jnp.sum((xb >= mid).astype(f32), axis=1, keepdims=True)
        ge = cnt >= k_eff
        return (jnp.where(ge, mid, lo), jnp.where(ge, hi, mid))

    lo, hi = jax.lax.fori_loop(
        0,
        32,
        bis,
        (jnp.zeros((8, 1), jnp.int32), jnp.full((8, 1), 0x7F800000, jnp.int32)),
    )
    gt = xb > lo
    cnt_gt = jnp.sum(gt.astype(f32), axis=1, keepdims=True)
    sum_gt = jnp.sum(jnp.where(gt, x, 0.0), axis=1, keepdims=True)
    t_val = jnp.max(jnp.where(xb == lo, x, 0.0), axis=1, keepdims=True)
    lc_row = sum_pos_ce + sum_gt + (k - cnt_gt) * t_val
    lc = jnp.sum(lc_row)
    npos = jnp.sum(npos_row)

    sub128 = jax.lax.broadcasted_iota(jnp.int32, (8, 128), 0)
    buf = jnp.where(
        sub128 == 0,
        ll,
        jnp.where(
            sub128 == 1,
            lc,
            jnp.where(sub128 == 2, llm, jnp.where(sub128 == 3, npos, 0.0)),
        ),
    )

    @pl.when(i == 0)
    def _():
        out_ref[...] = jnp.zeros((8, 128), f32)

    out_ref[...] += buf


def kernel(loc_data, conf_data, landm_data, priors, targets):
    num, P, _ = loc_data.shape
    PP = ((P + 127) // 128) * 128
    pad = PP - P
    f32 = jnp.float32
    B = 8

    locT = jnp.pad(loc_data.transpose(2, 0, 1), ((0, 0), (0, 0), (0, pad)))
    confT = jnp.pad(conf_data.transpose(2, 0, 1), ((0, 0), (0, 0), (0, pad)))
    landmT = jnp.pad(landm_data.transpose(2, 0, 1), ((0, 0), (0, 0), (0, pad)))
    pad_cols = jnp.concatenate(
        [jnp.zeros((2, pad), f32), jnp.ones((2, pad), f32)], axis=0
    )
    priorsT = jnp.concatenate([priors.T, pad_cols], axis=1)
    tgt = jnp.pad(targets, ((0, 0), (0, 0), (0, 1)))

    out = pl.pallas_call(
        functools.partial(_body, P),
        grid=(num // B,),
        in_specs=[
            pl.BlockSpec((4, B, PP), lambda i: (0, i, 0)),
            pl.BlockSpec((_NUM_CLASSES, B, PP), lambda i: (0, i, 0)),
            pl.BlockSpec((10, B, PP), lambda i: (0, i, 0)),
            pl.BlockSpec((4, PP), lambda i: (0, 0)),
            pl.BlockSpec((B, 8, 16), lambda i: (i, 0, 0)),
        ],
        out_specs=pl.BlockSpec((8, 128), lambda i: (0, 0)),
        out_shape=jax.ShapeDtypeStruct((8, 128), f32),
    )(locT, confT, landmT, priorsT, tgt)

    ll = out[0, 0]
    lc = out[1, 0]
    llm = out[2, 0]
    npos = out[3, 0]
    n = jnp.maximum(npos, 1.0)
    return ll / n, lc / n, llm / n
